# baseline (speedup 1.0000x reference)
import functools

import jax
import jax.numpy as jnp
from jax import lax
from jax.experimental import pallas as pl
from jax.experimental.pallas import tpu as pltpu
from jax.experimental.pallas import tpu_sc as plsc

N = 10000
H = 128
E = 320000
NC = 2
NS = 16
NW = NC * NS
EPW = E // NW
CHUNK = 80
NCHUNK = EPW // CHUNK

RPT = (N // NS) // 8 * 8
RTAIL = N - RPT * NS

assert EPW * NW == E and NCHUNK * CHUNK == EPW and RTAIL % 8 == 0


def _make_sc_agg():
    mesh = plsc.VectorSubcoreMesh(core_axis_name="c", subcore_axis_name="s",
                                  num_cores=NC, num_subcores=NS)

    @functools.partial(
        pl.kernel,
        out_type=jax.ShapeDtypeStruct((NC, N, H), jnp.float32),
        mesh=mesh,
        scratch_types=(
            [
                pltpu.VMEM_SHARED((N, H), jnp.float32),
                pltpu.VMEM((4, CHUNK, H), jnp.float32),
                pltpu.VMEM((8, CHUNK), jnp.int32),
                pltpu.VMEM((8, CHUNK), jnp.int32),
            ]
            + [pltpu.SemaphoreType.DMA for _ in range(25)]
        ),
    )
    def sc_agg(x_hbm, ei_hbm, out_hbm,
               acc_sh, rows_v, srcix, dstix, *rest):
        isem_s = rest[0:8]
        isem_d = rest[8:16]
        gsem = rest[16:20]
        ssem = rest[20:24]
        zsem = rest[24]
        c = lax.axis_index("c")
        s = lax.axis_index("s")
        wid = s * NC + c
        base = wid * EPW

        def start_idx(j, p):
            off = base + j * CHUNK
            pltpu.async_copy(ei_hbm.at[pl.ds(off, CHUNK)], srcix.at[p],
                             isem_s[p])
            pltpu.async_copy(ei_hbm.at[pl.ds(E + off, CHUNK)], dstix.at[p],
                             isem_d[p])

        def wait_idx(p):
            pltpu.make_async_copy(ei_hbm.at[pl.ds(0, CHUNK)], srcix.at[p],
                                  isem_s[p]).wait()
            pltpu.make_async_copy(ei_hbm.at[pl.ds(0, CHUNK)], dstix.at[p],
                                  isem_d[p]).wait()

        def start_gather(ip, rp):
            pltpu.async_copy(x_hbm.at[srcix.at[ip]], rows_v.at[rp], gsem[rp])

        def wait_gather(rp):
            pltpu.make_async_copy(x_hbm.at[srcix.at[0]], rows_v.at[rp],
                                  gsem[rp]).wait()

        def start_scatter(ip, rp):
            pltpu.async_copy(rows_v.at[rp], acc_sh.at[dstix.at[ip]], ssem[rp],
                             add=True)

        def wait_scatter(rp):
            pltpu.make_async_copy(rows_v.at[rp], acc_sh.at[dstix.at[0]],
                                  ssem[rp]).wait()

        zvec = jnp.zeros((16,), jnp.float32)

        def zero_body(i, carry):
            rows_v[0, i // 8, pl.ds((i % 8) * 16, 16)] = zvec
            return carry

        lax.fori_loop(0, CHUNK * 8, zero_body, 0, unroll=False)
        for k in range(7):
            pltpu.async_copy(rows_v.at[0],
                             acc_sh.at[pl.ds(s * RPT + k * CHUNK, CHUNK)],
                             zsem)
        pltpu.async_copy(rows_v.at[0].at[pl.ds(0, 64)],
                         acc_sh.at[pl.ds(s * RPT + 560, 64)], zsem)

        @pl.when(s == NS - 1)
        def _():
            pltpu.async_copy(rows_v.at[0].at[pl.ds(0, RTAIL)],
                             acc_sh.at[pl.ds(NS * RPT, RTAIL)], zsem)

        for p in range(7):
            start_idx(p, p)

        for k in range(7):
            pltpu.make_async_copy(
                rows_v.at[0], acc_sh.at[pl.ds(s * RPT, CHUNK)], zsem).wait()
        pltpu.make_async_copy(rows_v.at[0].at[pl.ds(0, 64)],
                              acc_sh.at[pl.ds(0, 64)], zsem).wait()

        @pl.when(s == NS - 1)
        def _():
            pltpu.make_async_copy(rows_v.at[0].at[pl.ds(0, RTAIL)],
                                  acc_sh.at[pl.ds(0, RTAIL)], zsem).wait()

        plsc.subcore_barrier()
        for p in range(3):
            wait_idx(p)
            start_gather(p, p)

        def body(j, ip, rp):
            @pl.when(j + 3 < NCHUNK)
            def _():
                wait_idx((ip + 3) % 8)

                @pl.when(j - 1 >= 0)
                def _():
                    wait_scatter((rp + 3) % 4)

                start_gather((ip + 3) % 8, (rp + 3) % 4)

            wait_gather(rp)
            start_scatter(ip, rp)

            @pl.when(j + 7 < NCHUNK)
            def _():
                start_idx(j + 7, (ip + 7) % 8)

        def outer(o, carry):
            for b in range(8):
                body(o * 8 + b, b, b % 4)
            return carry

        nmain = (NCHUNK // 8) * 8
        lax.fori_loop(0, NCHUNK // 8, outer, 0, unroll=False)
        for jj in range(nmain, NCHUNK):
            body(jnp.int32(jj), jj % 8, jj % 4)

        wait_scatter(0)
        wait_scatter(1)
        wait_scatter(2)
        wait_scatter(3)

        plsc.subcore_barrier()
        pltpu.sync_copy(acc_sh.at[pl.ds(s * RPT, RPT)],
                        out_hbm.at[c, pl.ds(s * RPT, RPT)])

        @pl.when(s == NS - 1)
        def _():
            pltpu.sync_copy(acc_sh.at[pl.ds(NS * RPT, RTAIL)],
                            out_hbm.at[c, pl.ds(NS * RPT, RTAIL)])

    return sc_agg


_SC_AGG_CACHE = []


def _sc_agg(*args):
    if not _SC_AGG_CACHE:
        _SC_AGG_CACHE.append(_make_sc_agg())
    return _SC_AGG_CACHE[0](*args)


def _tc_head_body(eps_ref, x_ref, agg_ref, w1_ref, b1_ref, g1_ref, be1_ref,
                  w2_ref, b2_ref, g4_ref, be4_ref, wl1_ref, bl1_ref,
                  wl3_ref, bl3_ref, out_ref):
    eps = eps_ref[0, 0]
    x = x_ref[...]
    agg = agg_ref[0] + agg_ref[1]

    h = (1.0 + eps) * x + agg
    h = jnp.dot(h.astype(jnp.bfloat16), w1_ref[...].astype(jnp.bfloat16),
                preferred_element_type=jnp.float32) + b1_ref[...]
    m = jnp.mean(h, axis=0, keepdims=True)
    v = jnp.mean(h * h, axis=0, keepdims=True) - m * m
    h = g1_ref[...] * (h - m) * lax.rsqrt(v + 1e-5) + be1_ref[...]
    h = jnp.maximum(h, 0.0)
    h = jnp.dot(h.astype(jnp.bfloat16), w2_ref[...].astype(jnp.bfloat16),
                preferred_element_type=jnp.float32) + b2_ref[...]
    h = jnp.where(h > 0, h, 0.01 * h)
    m4 = jnp.mean(h, axis=0, keepdims=True)
    v4 = jnp.mean(h * h, axis=0, keepdims=True) - m4 * m4
    h = g4_ref[...] * (h - m4) * lax.rsqrt(v4 + 1e-5) + be4_ref[...]
    h = jnp.where(h > 0, h, 0.1 * h)
    h = x + 0.01 * h
    h = jnp.dot(h, wl1_ref[...], preferred_element_type=jnp.float32) + bl1_ref[...]
    h = jnp.where(h > 0, h, 0.1 * h)
    out_ref[...] = (jnp.dot(h, wl3_ref[...], preferred_element_type=jnp.float32)
                    + bl3_ref[...])


def _tc_head(gin_eps, x, agg2, W1, b1, gamma1, beta1, W2, b2, gamma4, beta4,
             Wl1, bl1, Wl3, bl3):
    C = Wl3.shape[1]
    eps_arr = jnp.reshape(gin_eps, (1, 1))
    smem_spec = pl.BlockSpec(memory_space=pltpu.SMEM)
    return pl.pallas_call(
        _tc_head_body,
        out_shape=jax.ShapeDtypeStruct((N, C), jnp.float32),
        in_specs=[smem_spec] + [pl.BlockSpec(memory_space=pltpu.VMEM)] * 14,
        out_specs=pl.BlockSpec(memory_space=pltpu.VMEM),
    )(eps_arr, x, agg2,
      W1, jnp.reshape(b1, (1, H)), jnp.reshape(gamma1, (1, H)),
      jnp.reshape(beta1, (1, H)),
      W2, jnp.reshape(b2, (1, H)), jnp.reshape(gamma4, (1, H)),
      jnp.reshape(beta4, (1, H)),
      Wl1, jnp.reshape(bl1, (1, Wl1.shape[1])),
      Wl3, jnp.reshape(bl3, (1, C)))


def kernel(x, edge_index, gin_eps, W1, b1, gamma1, beta1, W2, b2,
           gamma4, beta4, Wl1, bl1, Wl3, bl3):
    ei = jnp.reshape(edge_index.astype(jnp.int32), (2 * E,))
    agg2 = _sc_agg(x, ei)
    return _tc_head(gin_eps, x, agg2, W1, b1, gamma1, beta1, W2, b2,
                    gamma4, beta4, Wl1, bl1, Wl3, bl3)


# speedup vs baseline: 1.0142x; 1.0142x over previous
import functools

import jax
import jax.numpy as jnp
from jax import lax
from jax.experimental import pallas as pl
from jax.experimental.pallas import tpu as pltpu
from jax.experimental.pallas import tpu_sc as plsc

N = 10000
H = 128
E = 320000
NC = 2
NS = 16
NW = NC * NS
EPW = E // NW
CHUNK = 80
NCHUNK = EPW // CHUNK

RPT = (N // NS) // 8 * 8
RTAIL = N - RPT * NS

assert EPW * NW == E and NCHUNK * CHUNK == EPW and RTAIL % 8 == 0


def _make_sc_agg():
    mesh = plsc.VectorSubcoreMesh(core_axis_name="c", subcore_axis_name="s",
                                  num_cores=NC, num_subcores=NS)

    @functools.partial(
        pl.kernel,
        out_type=jax.ShapeDtypeStruct((NC, N, H), jnp.float32),
        mesh=mesh,
        scratch_types=(
            [
                pltpu.VMEM_SHARED((N, H), jnp.float32),
                pltpu.VMEM((4, CHUNK, H), jnp.float32),
                pltpu.VMEM((8, CHUNK), jnp.int32),
                pltpu.VMEM((8, CHUNK), jnp.int32),
            ]
            + [pltpu.SemaphoreType.DMA for _ in range(25)]
        ),
    )
    def sc_agg(x_hbm, ei_hbm, out_hbm,
               acc_sh, rows_v, srcix, dstix, *rest):
        isem_s = rest[0:8]
        isem_d = rest[8:16]
        gsem = rest[16:20]
        ssem = rest[20:24]
        zsem = rest[24]
        c = lax.axis_index("c")
        s = lax.axis_index("s")
        wid = s * NC + c
        base = wid * EPW

        def start_idx(j, p):
            off = base + j * CHUNK
            pltpu.async_copy(ei_hbm.at[pl.ds(off, CHUNK)], srcix.at[p],
                             isem_s[p])
            pltpu.async_copy(ei_hbm.at[pl.ds(E + off, CHUNK)], dstix.at[p],
                             isem_d[p])

        def wait_idx(p):
            pltpu.make_async_copy(ei_hbm.at[pl.ds(0, CHUNK)], srcix.at[p],
                                  isem_s[p]).wait()
            pltpu.make_async_copy(ei_hbm.at[pl.ds(0, CHUNK)], dstix.at[p],
                                  isem_d[p]).wait()

        def start_gather(ip, rp):
            pltpu.async_copy(x_hbm.at[srcix.at[ip]], rows_v.at[rp], gsem[rp])

        def wait_gather(rp):
            pltpu.make_async_copy(x_hbm.at[srcix.at[0]], rows_v.at[rp],
                                  gsem[rp]).wait()

        def start_scatter(ip, rp):
            pltpu.async_copy(rows_v.at[rp], acc_sh.at[dstix.at[ip]], ssem[rp],
                             add=True)

        def wait_scatter(rp):
            pltpu.make_async_copy(rows_v.at[rp], acc_sh.at[dstix.at[0]],
                                  ssem[rp]).wait()

        zvec = jnp.zeros((16,), jnp.float32)

        def zero_body(i, carry):
            rows_v[0, i // 8, pl.ds((i % 8) * 16, 16)] = zvec
            return carry

        lax.fori_loop(0, CHUNK * 8, zero_body, 0, unroll=False)
        for k in range(7):
            pltpu.async_copy(rows_v.at[0],
                             acc_sh.at[pl.ds(s * RPT + k * CHUNK, CHUNK)],
                             zsem)
        pltpu.async_copy(rows_v.at[0].at[pl.ds(0, 64)],
                         acc_sh.at[pl.ds(s * RPT + 560, 64)], zsem)

        @pl.when(s == NS - 1)
        def _():
            pltpu.async_copy(rows_v.at[0].at[pl.ds(0, RTAIL)],
                             acc_sh.at[pl.ds(NS * RPT, RTAIL)], zsem)

        for p in range(7):
            start_idx(p, p)

        for k in range(7):
            pltpu.make_async_copy(
                rows_v.at[0], acc_sh.at[pl.ds(s * RPT, CHUNK)], zsem).wait()
        pltpu.make_async_copy(rows_v.at[0].at[pl.ds(0, 64)],
                              acc_sh.at[pl.ds(0, 64)], zsem).wait()

        @pl.when(s == NS - 1)
        def _():
            pltpu.make_async_copy(rows_v.at[0].at[pl.ds(0, RTAIL)],
                                  acc_sh.at[pl.ds(0, RTAIL)], zsem).wait()

        plsc.subcore_barrier()
        for p in range(3):
            wait_idx(p)
            start_gather(p, p)

        def body(j, ip, rp):
            @pl.when(j + 3 < NCHUNK)
            def _():
                wait_idx((ip + 3) % 8)

                @pl.when(j - 1 >= 0)
                def _():
                    wait_scatter((rp + 3) % 4)

                start_gather((ip + 3) % 8, (rp + 3) % 4)

            wait_gather(rp)
            start_scatter(ip, rp)

            @pl.when(j + 7 < NCHUNK)
            def _():
                start_idx(j + 7, (ip + 7) % 8)

        def outer(o, carry):
            for b in range(8):
                body(o * 8 + b, b, b % 4)
            return carry

        nmain = (NCHUNK // 8) * 8
        lax.fori_loop(0, NCHUNK // 8, outer, 0, unroll=False)
        for jj in range(nmain, NCHUNK):
            body(jnp.int32(jj), jj % 8, jj % 4)

        wait_scatter(0)
        wait_scatter(1)
        wait_scatter(2)
        wait_scatter(3)

        plsc.subcore_barrier()
        pltpu.sync_copy(acc_sh.at[pl.ds(s * RPT, RPT)],
                        out_hbm.at[c, pl.ds(s * RPT, RPT)])

        @pl.when(s == NS - 1)
        def _():
            pltpu.sync_copy(acc_sh.at[pl.ds(NS * RPT, RTAIL)],
                            out_hbm.at[c, pl.ds(NS * RPT, RTAIL)])

    return sc_agg


_SC_AGG_CACHE = []


def _sc_agg(*args):
    if not _SC_AGG_CACHE:
        _SC_AGG_CACHE.append(_make_sc_agg())
    return _SC_AGG_CACHE[0](*args)


def _tc_head_body(eps_ref, x_ref, agg_ref, w1_ref, b1_ref, g1_ref, be1_ref,
                  w2_ref, b2_ref, g4_ref, be4_ref, wl1_ref, bl1_ref,
                  wl3_ref, bl3_ref, out_ref):
    eps = eps_ref[0, 0]
    x = x_ref[...]
    agg = agg_ref[0] + agg_ref[1]

    h = (1.0 + eps) * x + agg
    h = jnp.dot(h, w1_ref[...], preferred_element_type=jnp.float32) + b1_ref[...]
    m = jnp.mean(h, axis=0, keepdims=True)
    v = jnp.mean(h * h, axis=0, keepdims=True) - m * m
    h = g1_ref[...] * (h - m) * lax.rsqrt(v + 1e-5) + be1_ref[...]
    h = jnp.maximum(h, 0.0)
    h = jnp.dot(h, w2_ref[...], preferred_element_type=jnp.float32) + b2_ref[...]
    h = jnp.where(h > 0, h, 0.01 * h)
    m4 = jnp.mean(h, axis=0, keepdims=True)
    v4 = jnp.mean(h * h, axis=0, keepdims=True) - m4 * m4
    h = g4_ref[...] * (h - m4) * lax.rsqrt(v4 + 1e-5) + be4_ref[...]
    h = jnp.where(h > 0, h, 0.1 * h)
    h = x + 0.01 * h
    h = jnp.dot(h, wl1_ref[...], preferred_element_type=jnp.float32) + bl1_ref[...]
    h = jnp.where(h > 0, h, 0.1 * h)
    out_ref[...] = (jnp.dot(h, wl3_ref[...], preferred_element_type=jnp.float32)
                    + bl3_ref[...])


def _tc_head(gin_eps, x, agg2, W1, b1, gamma1, beta1, W2, b2, gamma4, beta4,
             Wl1, bl1, Wl3, bl3):
    C = Wl3.shape[1]
    eps_arr = jnp.reshape(gin_eps, (1, 1))
    smem_spec = pl.BlockSpec(memory_space=pltpu.SMEM)
    return pl.pallas_call(
        _tc_head_body,
        out_shape=jax.ShapeDtypeStruct((N, C), jnp.float32),
        in_specs=[smem_spec] + [pl.BlockSpec(memory_space=pltpu.VMEM)] * 14,
        out_specs=pl.BlockSpec(memory_space=pltpu.VMEM),
    )(eps_arr, x, agg2,
      W1, jnp.reshape(b1, (1, H)), jnp.reshape(gamma1, (1, H)),
      jnp.reshape(beta1, (1, H)),
      W2, jnp.reshape(b2, (1, H)), jnp.reshape(gamma4, (1, H)),
      jnp.reshape(beta4, (1, H)),
      Wl1, jnp.reshape(bl1, (1, Wl1.shape[1])),
      Wl3, jnp.reshape(bl3, (1, C)))


def kernel(x, edge_index, gin_eps, W1, b1, gamma1, beta1, W2, b2,
           gamma4, beta4, Wl1, bl1, Wl3, bl3):
    ei = jnp.reshape(edge_index.astype(jnp.int32), (2 * E,))
    agg2 = _sc_agg(x, ei)
    return _tc_head(gin_eps, x, agg2, W1, b1, gamma1, beta1, W2, b2,
                    gamma4, beta4, Wl1, bl1, Wl3, bl3)
